# baseline (speedup 1.0000x reference)
import functools

import jax
import jax.numpy as jnp
from jax import lax
from jax.experimental import pallas as pl
from jax.experimental.pallas import tpu as pltpu
from jax.experimental.pallas import tpu_sc as plsc

N, E, D, H, C = 10000, 320000, 128, 96, 40
HP = 128
NC, NS = 2, 16
LANES = 16
CHUNK = 128
NCHUNK = 80
EPAD = NC * NS * NCHUNK * CHUNK
RPT = 640
NPAD = NS * RPT


@functools.partial(
    pl.kernel,
    out_type=jax.ShapeDtypeStruct((NC, NPAD, HP), jnp.float32),
    mesh=plsc.VectorSubcoreMesh(core_axis_name="c", subcore_axis_name="s"),
    compiler_params=pltpu.CompilerParams(needs_layout_passes=False),
    scratch_types=[
        pltpu.VMEM((2, CHUNK), jnp.int32),
        pltpu.VMEM((2, CHUNK), jnp.int32),
        pltpu.VMEM((CHUNK, HP), jnp.float32),
        pltpu.VMEM((CHUNK, HP), jnp.float32),
        pltpu.VMEM((64, HP), jnp.float32),
        pltpu.VMEM_SHARED((NPAD, HP), jnp.float32),
        pltpu.SemaphoreType.DMA,
        pltpu.SemaphoreType.DMA,
        pltpu.SemaphoreType.DMA,
        pltpu.SemaphoreType.DMA,
        pltpu.SemaphoreType.DMA,
        pltpu.SemaphoreType.DMA,
    ],
)
def _agg(h_hbm, idx_hbm, out_hbm, ixb0, ixb1, rows0_v, rows1_v,
         zbuf_v, agg_s, isem0, isem1, gsem0, gsem1, ssem0, ssem1):
    c = lax.axis_index("c")
    s = lax.axis_index("s")

    zeros = jnp.zeros((LANES,), jnp.float32)
    for i in range(64):
        for j in range(HP // LANES):
            zbuf_v[i, pl.ds(j * LANES, LANES)] = zeros

    def zslab(k, carry):
        pltpu.sync_copy(zbuf_v, agg_s.at[pl.ds(s * RPT + k * 64, 64)])
        return carry

    lax.fori_loop(0, RPT // 64, zslab, 0)

    ixbs = (ixb0, ixb1)
    rbs = (rows0_v, rows1_v)
    isems = (isem0, isem1)
    gsems = (gsem0, gsem1)
    ssems = (ssem0, ssem1)

    def ix_start(j, b):
        pltpu.async_copy(idx_hbm.at[c, s, j], ixbs[b], isems[b])

    def ix_wait(j, b):
        pltpu.make_async_copy(idx_hbm.at[c, s, j], ixbs[b], isems[b]).wait()

    def g_start(b):
        pltpu.async_copy(h_hbm.at[ixbs[b].at[0]], rbs[b], gsems[b])

    def g_wait(b):
        pltpu.make_async_copy(h_hbm.at[ixbs[b].at[0]], rbs[b],
                              gsems[b]).wait()

    def s_start(b):
        pltpu.async_copy(rbs[b], agg_s.at[ixbs[b].at[1]], ssems[b], add=True)

    def s_wait(b):
        pltpu.make_async_copy(rbs[b], agg_s.at[ixbs[b].at[1]],
                              ssems[b]).wait()

    plsc.subcore_barrier()

    ix_start(0, 0)
    ix_wait(0, 0)
    g_start(0)
    g_wait(0)
    ix_start(1, 1)
    s_start(0)
    ix_wait(1, 1)
    g_start(1)
    g_wait(1)
    s_wait(0)
    ix_start(2, 0)
    s_start(1)

    def steady(t, carry):
        for b in range(2):
            j = 2 * t + 2 + b
            ix_wait(j, b)
            g_start(b)
            g_wait(b)
            s_wait(1 - b)
            ix_start(j + 1, 1 - b)
            s_start(b)
        return carry

    lax.fori_loop(0, (NCHUNK - 4) // 2, steady, 0)

    ix_wait(NCHUNK - 2, 0)
    g_start(0)
    g_wait(0)
    s_wait(1)
    ix_start(NCHUNK - 1, 1)
    s_start(0)
    ix_wait(NCHUNK - 1, 1)
    g_start(1)
    g_wait(1)
    s_wait(0)
    s_start(1)
    s_wait(1)
    plsc.subcore_barrier()

    pltpu.sync_copy(
        agg_s.at[pl.ds(s * RPT, RPT)],
        out_hbm.at[c, pl.ds(s * RPT, RPT)])


def _zpad(u):
    return jnp.concatenate(
        [u, jnp.zeros((N, HP - H), jnp.float32)], axis=1)


def _agg_sum(h_ref, p_ref):
    return h_ref[:, :H] + p_ref[0, :N, :H] + p_ref[1, :N, :H]


def _emb_body(x_ref, w_ref, b_ref, o_ref):
    u = (jnp.dot(x_ref[...], w_ref[...], preferred_element_type=jnp.float32)
         + b_ref[...])
    o_ref[...] = _zpad(u)


def _bn_relu(u, g, be):
    mean = jnp.mean(u, axis=0, keepdims=True)
    var = jnp.mean((u - mean) ** 2, axis=0, keepdims=True)
    return jnp.maximum(g * (u - mean) * lax.rsqrt(var + 1e-5) + be, 0.0)


def _gin_body(h_ref, p_ref, w_ref, b_ref, g_ref, be_ref, o_ref):
    t = _agg_sum(h_ref, p_ref)
    u = (jnp.dot(t, w_ref[...], preferred_element_type=jnp.float32)
         + b_ref[...])
    o_ref[...] = _zpad(_bn_relu(u, g_ref[...], be_ref[...]))


def _fin_body(h_ref, p_ref, w_ref, b_ref, g_ref, be_ref, wro_ref, bro_ref,
              o_ref):
    t = _agg_sum(h_ref, p_ref)
    u = (jnp.dot(t, w_ref[...], preferred_element_type=jnp.float32)
         + b_ref[...])
    h2 = _bn_relu(u, g_ref[...], be_ref[...])
    z = (jnp.dot(h2, wro_ref[...], preferred_element_type=jnp.float32)
         + bro_ref[...])
    z = z - jnp.max(z, axis=1, keepdims=True)
    o_ref[...] = z - jnp.log(jnp.sum(jnp.exp(z), axis=1, keepdims=True))


_emb = pl.pallas_call(
    _emb_body, out_shape=jax.ShapeDtypeStruct((N, HP), jnp.float32))
_gin = pl.pallas_call(
    _gin_body, out_shape=jax.ShapeDtypeStruct((N, HP), jnp.float32))
_fin = pl.pallas_call(
    _fin_body, out_shape=jax.ShapeDtypeStruct((N, C), jnp.float32))


def kernel(x, edge_index, W_emb, b_emb, W1, b1, g1, be1, W2, b2, g2, be2,
           W_ro, b_ro):
    npad_e = EPAD - E
    src = jnp.concatenate(
        [edge_index[0], jnp.zeros((npad_e,), jnp.int32)])
    dst = jnp.concatenate(
        [edge_index[1], jnp.full((npad_e,), NPAD - 1, jnp.int32)])
    idx5 = jnp.stack(
        [src.reshape(NC, NS, NCHUNK, CHUNK),
         dst.reshape(NC, NS, NCHUNK, CHUNK)], axis=3)
    h0 = _emb(x, W_emb, b_emb.reshape(1, H))
    p = _agg(h0, idx5)
    h1 = _gin(h0, p, W1, b1.reshape(1, H), g1.reshape(1, H),
              be1.reshape(1, H))
    p = _agg(h1, idx5)
    return _fin(h1, p, W2, b2.reshape(1, H), g2.reshape(1, H),
                be2.reshape(1, H), W_ro, b_ro.reshape(1, C))


# speedup vs baseline: 1.5045x; 1.5045x over previous
import functools

import jax
import jax.numpy as jnp
from jax import lax
from jax.experimental import pallas as pl
from jax.experimental.pallas import tpu as pltpu
from jax.experimental.pallas import tpu_sc as plsc

N, E, D, H, C = 10000, 320000, 128, 96, 40
HP = 128
NC, NS = 2, 16
LANES = 16
CHUNK = 64
NCHUNK = 158
EPAD = NC * NS * NCHUNK * CHUNK
RPT = 640
NPAD = NS * RPT


@functools.partial(
    pl.kernel,
    out_type=jax.ShapeDtypeStruct((NC, NPAD, HP), jnp.float32),
    mesh=plsc.VectorSubcoreMesh(core_axis_name="c", subcore_axis_name="s"),
    compiler_params=pltpu.CompilerParams(needs_layout_passes=False),
    scratch_types=[
        pltpu.VMEM((NCHUNK, 2 * CHUNK), jnp.int32),
        pltpu.VMEM((CHUNK, HP), jnp.float32),
        pltpu.VMEM((CHUNK, HP), jnp.float32),
        pltpu.VMEM((8, HP), jnp.float32),
        pltpu.VMEM_SHARED((NPAD, HP), jnp.float32),
        pltpu.SemaphoreType.DMA,
        pltpu.SemaphoreType.DMA,
        pltpu.SemaphoreType.DMA,
        pltpu.SemaphoreType.DMA,
    ],
)
def _agg(h_hbm, idx_hbm, out_hbm, ixs_v, rows0_v, rows1_v,
         zbuf_v, agg_s, gsem0, gsem1, ssem0, ssem1):
    c = lax.axis_index("c")
    s = lax.axis_index("s")

    pltpu.sync_copy(idx_hbm.at[c, s], ixs_v)

    zeros = jnp.zeros((LANES,), jnp.float32)
    for i in range(8):
        for j in range(HP // LANES):
            zbuf_v[i, pl.ds(j * LANES, LANES)] = zeros

    def zslab(k, carry):
        pltpu.sync_copy(zbuf_v, agg_s.at[pl.ds(s * RPT + k * 8, 8)])
        return carry

    lax.fori_loop(0, RPT // 8, zslab, 0)

    rbs = (rows0_v, rows1_v)
    gsems = (gsem0, gsem1)
    ssems = (ssem0, ssem1)

    def g_start(j, b):
        pltpu.async_copy(
            h_hbm.at[ixs_v.at[j, pl.ds(CHUNK, CHUNK)]], rbs[b], gsems[b])

    def g_wait(j, b):
        pltpu.make_async_copy(
            h_hbm.at[ixs_v.at[j, pl.ds(CHUNK, CHUNK)]], rbs[b],
            gsems[b]).wait()

    def s_start(j, b):
        pltpu.async_copy(
            rbs[b], agg_s.at[ixs_v.at[j, pl.ds(0, CHUNK)]], ssems[b],
            add=True)

    def s_wait(j, b):
        pltpu.make_async_copy(
            rbs[b], agg_s.at[ixs_v.at[j, pl.ds(0, CHUNK)]],
            ssems[b]).wait()

    plsc.subcore_barrier()

    g_start(0, 0)
    g_wait(0, 0)
    g_start(1, 1)
    s_start(0, 0)

    def steady(t, carry):
        for b in range(2):
            j = 2 * t + 1 + b
            g_wait(j, b ^ 1)
            s_wait(j - 1, b)
            g_start(j + 1, b)
            s_start(j, b ^ 1)
        return carry

    lax.fori_loop(0, (NCHUNK - 2) // 2, steady, 0)

    g_wait(NCHUNK - 1, 1)
    s_wait(NCHUNK - 2, 0)
    s_start(NCHUNK - 1, 1)
    s_wait(NCHUNK - 1, 1)
    plsc.subcore_barrier()

    pltpu.sync_copy(
        agg_s.at[pl.ds(s * RPT, RPT)],
        out_hbm.at[c, pl.ds(s * RPT, RPT)])


def _zpad(u):
    return jnp.concatenate(
        [u, jnp.zeros((N, HP - H), jnp.float32)], axis=1)


def _agg_sum(h_ref, p_ref):
    return h_ref[:, :H] + p_ref[0, :N, :H] + p_ref[1, :N, :H]


def _emb_body(x_ref, w_ref, b_ref, o_ref):
    u = (jnp.dot(x_ref[...], w_ref[...], preferred_element_type=jnp.float32)
         + b_ref[...])
    o_ref[...] = _zpad(u)


def _bn_relu(u, g, be):
    mean = jnp.mean(u, axis=0, keepdims=True)
    var = jnp.mean((u - mean) ** 2, axis=0, keepdims=True)
    return jnp.maximum(g * (u - mean) * lax.rsqrt(var + 1e-5) + be, 0.0)


def _gin_body(h_ref, p_ref, w_ref, b_ref, g_ref, be_ref, o_ref):
    t = _agg_sum(h_ref, p_ref)
    u = (jnp.dot(t, w_ref[...], preferred_element_type=jnp.float32)
         + b_ref[...])
    o_ref[...] = _zpad(_bn_relu(u, g_ref[...], be_ref[...]))


def _fin_body(h_ref, p_ref, w_ref, b_ref, g_ref, be_ref, wro_ref, bro_ref,
              o_ref):
    t = _agg_sum(h_ref, p_ref)
    u = (jnp.dot(t, w_ref[...], preferred_element_type=jnp.float32)
         + b_ref[...])
    h2 = _bn_relu(u, g_ref[...], be_ref[...])
    z = (jnp.dot(h2, wro_ref[...], preferred_element_type=jnp.float32)
         + bro_ref[...])
    z = z - jnp.max(z, axis=1, keepdims=True)
    o_ref[...] = z - jnp.log(jnp.sum(jnp.exp(z), axis=1, keepdims=True))


_emb = pl.pallas_call(
    _emb_body, out_shape=jax.ShapeDtypeStruct((N, HP), jnp.float32))
_gin = pl.pallas_call(
    _gin_body, out_shape=jax.ShapeDtypeStruct((N, HP), jnp.float32))
_fin = pl.pallas_call(
    _fin_body, out_shape=jax.ShapeDtypeStruct((N, C), jnp.float32))


def kernel(x, edge_index, W_emb, b_emb, W1, b1, g1, be1, W2, b2, g2, be2,
           W_ro, b_ro):
    npad_e = EPAD - E
    src = jnp.concatenate(
        [edge_index[0], jnp.zeros((npad_e,), jnp.int32)])
    dst = jnp.concatenate(
        [edge_index[1], jnp.full((npad_e,), NPAD - 1, jnp.int32)])
    idx5 = jnp.concatenate(
        [dst.reshape(NC, NS, NCHUNK, CHUNK),
         src.reshape(NC, NS, NCHUNK, CHUNK)], axis=3)
    h0 = _emb(x, W_emb, b_emb.reshape(1, H))
    p = _agg(h0, idx5)
    h1 = _gin(h0, p, W1, b1.reshape(1, H), g1.reshape(1, H),
              be1.reshape(1, H))
    p = _agg(h1, idx5)
    return _fin(h1, p, W2, b2.reshape(1, H), g2.reshape(1, H),
                be2.reshape(1, H), W_ro, b_ro.reshape(1, C))
